# baseline (speedup 1.0000x reference)
import functools

import jax
import jax.numpy as jnp
from jax import lax
from jax.experimental import pallas as pl
from jax.experimental.pallas import tpu as pltpu
from jax.experimental.pallas import tpu_sc as plsc

N_NODES = 10000
N_EDGES = 320000
D = 128
E_DIM = 16

NC = 2
NS = 16
B = 80
EDGES_PER_TILE = N_EDGES // (NC * NS)
CHUNKS = EDGES_PER_TILE // B
ROWS_PER_TILE = 624
TAIL_ROWS = N_NODES - NS * ROWS_PER_TILE


def _node_msgs_tc(node_feats, W_nodeT):
    BLK = 2000

    def body(x_ref, w_ref, o_ref):
        o_ref[...] = jnp.dot(x_ref[...], w_ref[...],
                             preferred_element_type=jnp.float32)

    return pl.pallas_call(
        body,
        grid=(N_NODES // BLK,),
        in_specs=[pl.BlockSpec((BLK, D), lambda i: (i, 0)),
                  pl.BlockSpec((D, D), lambda i: (0, 0))],
        out_specs=pl.BlockSpec((BLK, D), lambda i: (i, 0)),
        out_shape=jax.ShapeDtypeStruct((N_NODES, D), jnp.float32),
    )(node_feats, W_nodeT)


def _edge_part_tc(edge_feats_t, W_edgeT, b_msg, off, n):
    BLK = 12800
    blk0 = off // BLK

    def body(x_ref, w_ref, b_ref, o_ref):
        y = jax.lax.dot_general(
            x_ref[...], w_ref[...],
            dimension_numbers=(((0,), (0,)), ((), ())),
            preferred_element_type=jnp.float32)
        o_ref[...] = y + b_ref[...]

    return pl.pallas_call(
        body,
        grid=(n // BLK,),
        in_specs=[pl.BlockSpec((E_DIM, BLK), lambda i: (0, i + blk0)),
                  pl.BlockSpec((E_DIM, D), lambda i: (0, 0)),
                  pl.BlockSpec((1, D), lambda i: (0, 0))],
        out_specs=pl.BlockSpec((BLK, D), lambda i: (i, 0)),
        out_shape=jax.ShapeDtypeStruct((n, D), jnp.float32),
    )(edge_feats_t, W_edgeT, b_msg.reshape(1, D))


def _sc_aggregate(node_msgs, src_idx, dst_idx, edge_part, eoff, per_tile):
    chunks = per_tile // B
    mesh = plsc.VectorSubcoreMesh(core_axis_name="c", subcore_axis_name="s")

    @functools.partial(
        pl.kernel,
        mesh=mesh,
        out_type=jax.ShapeDtypeStruct((NC * N_NODES, D), jnp.float32),
        scratch_types=[
            pltpu.VMEM((2, B), jnp.int32),
            pltpu.VMEM((2, B), jnp.int32),
            pltpu.VMEM((B, D), jnp.float32),
            pltpu.VMEM((B, D), jnp.float32),
            pltpu.VMEM((B, D), jnp.float32),
            pltpu.VMEM((B, D), jnp.float32),
            pltpu.VMEM_SHARED((N_NODES, D), jnp.float32),
            pltpu.SemaphoreType.DMA,
            pltpu.SemaphoreType.DMA,
            pltpu.SemaphoreType.DMA,
            pltpu.SemaphoreType.DMA,
            pltpu.SemaphoreType.DMA,
        ],
    )
    def k(msgs_hbm, src_hbm, dst_hbm, ep_hbm, out_hbm,
          src_v, dst_v, rows_a, rows_b, ep_a, ep_b, agg_sh,
          gsem, esem, ssem, sisem, disem):
        cid = lax.axis_index("c")
        sid = lax.axis_index("s")
        wid = cid * NS + sid
        zeros = jnp.zeros((16,), jnp.float32)

        def zrow(r, _):
            for j in range(D // 16):
                ep_a[r, pl.ds(j * 16, 16)] = zeros
            return 0
        lax.fori_loop(0, B, zrow, 0)

        row0 = sid * ROWS_PER_TILE
        n_full = ROWS_PER_TILE // B
        rem = ROWS_PER_TILE - n_full * B

        def zcopy(i, _):
            pltpu.sync_copy(ep_a, agg_sh.at[pl.ds(row0 + i * B, B)])
            return 0
        lax.fori_loop(0, n_full, zcopy, 0)
        if rem:
            pltpu.sync_copy(ep_a.at[pl.ds(0, rem)],
                            agg_sh.at[pl.ds(row0 + n_full * B, rem)])

        @pl.when(sid == NS - 1)
        def _():
            pltpu.sync_copy(ep_a.at[pl.ds(0, TAIL_ROWS)],
                            agg_sh.at[pl.ds(NS * ROWS_PER_TILE, TAIL_ROWS)])
        plsc.subcore_barrier()

        base_g = eoff + wid * per_tile
        base_l = wid * per_tile

        def issue_src(c, p):
            pltpu.async_copy(src_hbm.at[pl.ds(base_g + c * B, B)],
                             src_v.at[p], sisem)

        def wait_src(p):
            pltpu.make_async_copy(src_hbm.at[pl.ds(base_g, B)],
                                  src_v.at[p], sisem).wait()

        def issue_dst(c, p):
            pltpu.async_copy(dst_hbm.at[pl.ds(base_g + c * B, B)],
                             dst_v.at[p], disem)

        def wait_dst(p):
            pltpu.make_async_copy(dst_hbm.at[pl.ds(base_g, B)],
                                  dst_v.at[p], disem).wait()

        def issue_pf(c, rows_v_, ep_v_, p):
            pltpu.async_copy(ep_hbm.at[pl.ds(base_l + c * B, B)], ep_v_,
                             esem)
            pltpu.async_copy(msgs_hbm.at[src_v.at[p]], rows_v_, gsem)

        def wait_pf(rows_v_, ep_v_, p):
            pltpu.make_async_copy(ep_hbm.at[pl.ds(base_l, B)],
                                  ep_v_, esem).wait()
            pltpu.make_async_copy(msgs_hbm.at[src_v.at[p]], rows_v_,
                                  gsem).wait()

        def compute(rows_v_, ep_v_):
            U = 4

            def crow(r0, _):
                for u in range(U):
                    r = r0 * U + u
                    for j in range(D // 16):
                        s = pl.ds(j * 16, 16)
                        ep_v_[r, s] = jnp.maximum(
                            ep_v_[r, s] + rows_v_[r, s], zeros)
                return 0
            lax.fori_loop(0, B // U, crow, 0)

        def issue_scatter(ep_v_, p):
            pltpu.async_copy(ep_v_, agg_sh.at[dst_v.at[p]], ssem, add=True)

        def wait_scatter(ep_v_, p):
            pltpu.make_async_copy(ep_v_, agg_sh.at[dst_v.at[p]],
                                  ssem).wait()

        issue_src(0, 0)
        wait_src(0)
        issue_pf(0, rows_a, ep_a, 0)
        issue_src(1, 1)
        issue_dst(0, 0)

        def step(c, p, rows_c, ep_c, rows_n, ep_n, last):
            wait_pf(rows_c, ep_c, p)

            @pl.when(c > 0)
            def _():
                wait_scatter(ep_n, 1 - p)
            if not last:
                issue_dst(c + 1, 1 - p)
                wait_src(1 - p)
                issue_pf(c + 1, rows_n, ep_n, 1 - p)

                @pl.when(c + 2 < chunks)
                def _():
                    issue_src(c + 2, p)
            compute(rows_c, ep_c)
            wait_dst(p)
            issue_scatter(ep_c, p)

        pairs = (chunks - 1) // 2

        def body(g, _):
            c0 = 2 * g
            step(c0, 0, rows_a, ep_a, rows_b, ep_b, False)
            step(c0 + 1, 1, rows_b, ep_b, rows_a, ep_a, False)
            return 0
        lax.fori_loop(0, pairs, body, 0)

        if chunks - 2 * pairs == 1:
            step(chunks - 1, 0, rows_a, ep_a, rows_b, ep_b, True)
            wait_scatter(ep_a, 0)
        else:
            step(chunks - 2, 0, rows_a, ep_a, rows_b, ep_b, False)
            step(chunks - 1, 1, rows_b, ep_b, rows_a, ep_a, True)
            wait_scatter(ep_b, 1)

        plsc.subcore_barrier()
        pltpu.sync_copy(agg_sh.at[pl.ds(row0, ROWS_PER_TILE)],
                        out_hbm.at[pl.ds(cid * N_NODES + row0, ROWS_PER_TILE)])

        @pl.when(sid == NS - 1)
        def _():
            t0 = NS * ROWS_PER_TILE
            pltpu.sync_copy(agg_sh.at[pl.ds(t0, TAIL_ROWS)],
                            out_hbm.at[pl.ds(cid * N_NODES + t0, TAIL_ROWS)])

    return k(node_msgs, src_idx, dst_idx, edge_part)


E_SPLITS = (64000, 128000, 128000)


def _final_tc(node_feats, aggs_list, W_out1T, W_out2T, b_out):
    BLK = 2000
    NB = N_NODES // BLK
    n_parts = 2 * len(aggs_list)

    def body(x_ref, *refs):
        a_refs = refs[:n_parts]
        w1_ref, w2_ref, b_ref, o_ref = refs[n_parts:]
        acc = jnp.dot(x_ref[...], w1_ref[...],
                      preferred_element_type=jnp.float32)
        agg = a_refs[0][...]
        for a in a_refs[1:]:
            agg = agg + a[...]
        acc = acc + jnp.dot(agg, w2_ref[...],
                            preferred_element_type=jnp.float32)
        o_ref[...] = jnp.maximum(acc + b_ref[...], 0.0)

    agg_specs = []
    agg_args = []
    for aggs in aggs_list:
        agg_specs.append(pl.BlockSpec((BLK, D), lambda i: (i, 0)))
        agg_specs.append(pl.BlockSpec((BLK, D), lambda i: (i + NB, 0)))
        agg_args.extend([aggs, aggs])

    return pl.pallas_call(
        body,
        grid=(NB,),
        in_specs=[pl.BlockSpec((BLK, D), lambda i: (i, 0))] + agg_specs +
                 [pl.BlockSpec((D, D), lambda i: (0, 0)),
                  pl.BlockSpec((D, D), lambda i: (0, 0)),
                  pl.BlockSpec((1, D), lambda i: (0, 0))],
        out_specs=pl.BlockSpec((BLK, D), lambda i: (i, 0)),
        out_shape=jax.ShapeDtypeStruct((N_NODES, D), jnp.float32),
    )(node_feats, *agg_args, W_out1T, W_out2T, b_out.reshape(1, D))


def kernel(node_feats, edge_index, edge_feats, W_msg, b_msg, W_out, b_out):
    edge_index = edge_index.astype(jnp.int32)
    src = edge_index[0]
    dst = edge_index[1]
    W_nodeT = W_msg[:, :D].T
    W_edgeT = W_msg[:, D:].T
    W_out1T = W_out[:, :D].T
    W_out2T = W_out[:, D:].T

    node_msgs = _node_msgs_tc(node_feats, W_nodeT)
    ef_t = edge_feats.T
    aggs_list = []
    off = 0
    for n_e in E_SPLITS:
        ep = _edge_part_tc(ef_t, W_edgeT, b_msg, off, n_e)
        aggs_list.append(
            _sc_aggregate(node_msgs, src, dst, ep, off, n_e // (NC * NS)))
        off += n_e
    return _final_tc(node_feats, aggs_list, W_out1T, W_out2T, b_out)


# speedup vs baseline: 1.0456x; 1.0456x over previous
import functools

import jax
import jax.numpy as jnp
from jax import lax
from jax.experimental import pallas as pl
from jax.experimental.pallas import tpu as pltpu
from jax.experimental.pallas import tpu_sc as plsc

N_NODES = 10000
N_EDGES = 320000
D = 128
E_DIM = 16

NC = 2
NS = 16
B = 80
EDGES_PER_TILE = N_EDGES // (NC * NS)
CHUNKS = EDGES_PER_TILE // B
ROWS_PER_TILE = 624
TAIL_ROWS = N_NODES - NS * ROWS_PER_TILE


def _node_msgs_tc(node_feats, W_nodeT):
    BLK = 2000

    def body(x_ref, w_ref, o_ref):
        o_ref[...] = jnp.dot(x_ref[...], w_ref[...],
                             preferred_element_type=jnp.float32)

    return pl.pallas_call(
        body,
        grid=(N_NODES // BLK,),
        in_specs=[pl.BlockSpec((BLK, D), lambda i: (i, 0)),
                  pl.BlockSpec((D, D), lambda i: (0, 0))],
        out_specs=pl.BlockSpec((BLK, D), lambda i: (i, 0)),
        out_shape=jax.ShapeDtypeStruct((N_NODES, D), jnp.float32),
    )(node_feats, W_nodeT)


def _edge_part_tc(edge_feats_t, W_edgeT, b_msg, off, n):
    BLK = 12800
    blk0 = off // BLK

    def body(x_ref, w_ref, b_ref, o_ref):
        y = jax.lax.dot_general(
            x_ref[...], w_ref[...],
            dimension_numbers=(((0,), (0,)), ((), ())),
            preferred_element_type=jnp.float32)
        o_ref[...] = y + b_ref[...]

    return pl.pallas_call(
        body,
        grid=(n // BLK,),
        in_specs=[pl.BlockSpec((E_DIM, BLK), lambda i: (0, i + blk0)),
                  pl.BlockSpec((E_DIM, D), lambda i: (0, 0)),
                  pl.BlockSpec((1, D), lambda i: (0, 0))],
        out_specs=pl.BlockSpec((BLK, D), lambda i: (i, 0)),
        out_shape=jax.ShapeDtypeStruct((n, D), jnp.float32),
    )(edge_feats_t, W_edgeT, b_msg.reshape(1, D))


def _sc_aggregate(node_msgs, src_idx, dst_idx, edge_part, eoff, per_tile):
    chunks = per_tile // B
    mesh = plsc.VectorSubcoreMesh(core_axis_name="c", subcore_axis_name="s")

    @functools.partial(
        pl.kernel,
        mesh=mesh,
        out_type=jax.ShapeDtypeStruct((NC * N_NODES, D), jnp.float32),
        scratch_types=[
            pltpu.VMEM((2, B), jnp.int32),
            pltpu.VMEM((2, B), jnp.int32),
            pltpu.VMEM((B, D), jnp.float32),
            pltpu.VMEM((B, D), jnp.float32),
            pltpu.VMEM((B, D), jnp.float32),
            pltpu.VMEM((B, D), jnp.float32),
            pltpu.VMEM_SHARED((N_NODES, D), jnp.float32),
            pltpu.SemaphoreType.DMA,
            pltpu.SemaphoreType.DMA,
            pltpu.SemaphoreType.DMA,
            pltpu.SemaphoreType.DMA,
            pltpu.SemaphoreType.DMA,
        ],
    )
    def k(msgs_hbm, src_hbm, dst_hbm, ep_hbm, out_hbm,
          src_v, dst_v, rows_a, rows_b, ep_a, ep_b, agg_sh,
          gsem, esem, ssem, sisem, disem):
        cid = lax.axis_index("c")
        sid = lax.axis_index("s")
        wid = cid * NS + sid
        zeros = jnp.zeros((16,), jnp.float32)

        def zrow(r, _):
            for j in range(D // 16):
                ep_a[r, pl.ds(j * 16, 16)] = zeros
            return 0
        lax.fori_loop(0, B, zrow, 0)

        row0 = sid * ROWS_PER_TILE
        n_full = ROWS_PER_TILE // B
        rem = ROWS_PER_TILE - n_full * B

        def zcopy(i, _):
            pltpu.sync_copy(ep_a, agg_sh.at[pl.ds(row0 + i * B, B)])
            return 0
        lax.fori_loop(0, n_full, zcopy, 0)
        if rem:
            pltpu.sync_copy(ep_a.at[pl.ds(0, rem)],
                            agg_sh.at[pl.ds(row0 + n_full * B, rem)])

        @pl.when(sid == NS - 1)
        def _():
            pltpu.sync_copy(ep_a.at[pl.ds(0, TAIL_ROWS)],
                            agg_sh.at[pl.ds(NS * ROWS_PER_TILE, TAIL_ROWS)])
        plsc.subcore_barrier()

        base_g = eoff + wid * per_tile
        base_l = wid * per_tile

        def issue_src(c, p):
            pltpu.async_copy(src_hbm.at[pl.ds(base_g + c * B, B)],
                             src_v.at[p], sisem)

        def wait_src(p):
            pltpu.make_async_copy(src_hbm.at[pl.ds(base_g, B)],
                                  src_v.at[p], sisem).wait()

        def issue_dst(c, p):
            pltpu.async_copy(dst_hbm.at[pl.ds(base_g + c * B, B)],
                             dst_v.at[p], disem)

        def wait_dst(p):
            pltpu.make_async_copy(dst_hbm.at[pl.ds(base_g, B)],
                                  dst_v.at[p], disem).wait()

        def issue_pf(c, rows_v_, ep_v_, p):
            pltpu.async_copy(ep_hbm.at[pl.ds(base_l + c * B, B)], ep_v_,
                             esem)
            pltpu.async_copy(msgs_hbm.at[src_v.at[p]], rows_v_, gsem)

        def wait_pf(rows_v_, ep_v_, p):
            pltpu.make_async_copy(ep_hbm.at[pl.ds(base_l, B)],
                                  ep_v_, esem).wait()
            pltpu.make_async_copy(msgs_hbm.at[src_v.at[p]], rows_v_,
                                  gsem).wait()

        def compute(rows_v_, ep_v_):
            U = 4

            def crow(r0, _):
                for u in range(U):
                    r = r0 * U + u
                    for j in range(D // 16):
                        s = pl.ds(j * 16, 16)
                        ep_v_[r, s] = jnp.maximum(
                            ep_v_[r, s] + rows_v_[r, s], zeros)
                return 0
            lax.fori_loop(0, B // U, crow, 0)

        def issue_scatter(ep_v_, p):
            pltpu.async_copy(ep_v_, agg_sh.at[dst_v.at[p]], ssem, add=True)

        def wait_scatter(ep_v_, p):
            pltpu.make_async_copy(ep_v_, agg_sh.at[dst_v.at[p]],
                                  ssem).wait()

        issue_src(0, 0)
        wait_src(0)
        issue_pf(0, rows_a, ep_a, 0)
        issue_src(1, 1)
        issue_dst(0, 0)

        def step(c, p, rows_c, ep_c, rows_n, ep_n, last):
            wait_pf(rows_c, ep_c, p)

            @pl.when(c > 0)
            def _():
                wait_scatter(ep_n, 1 - p)
            if not last:
                issue_dst(c + 1, 1 - p)
                wait_src(1 - p)
                issue_pf(c + 1, rows_n, ep_n, 1 - p)

                @pl.when(c + 2 < chunks)
                def _():
                    issue_src(c + 2, p)
            compute(rows_c, ep_c)
            wait_dst(p)
            issue_scatter(ep_c, p)

        pairs = (chunks - 1) // 2

        def body(g, _):
            c0 = 2 * g
            step(c0, 0, rows_a, ep_a, rows_b, ep_b, False)
            step(c0 + 1, 1, rows_b, ep_b, rows_a, ep_a, False)
            return 0
        lax.fori_loop(0, pairs, body, 0)

        if chunks - 2 * pairs == 1:
            step(chunks - 1, 0, rows_a, ep_a, rows_b, ep_b, True)
            wait_scatter(ep_a, 0)
        else:
            step(chunks - 2, 0, rows_a, ep_a, rows_b, ep_b, False)
            step(chunks - 1, 1, rows_b, ep_b, rows_a, ep_a, True)
            wait_scatter(ep_b, 1)

        plsc.subcore_barrier()
        pltpu.sync_copy(agg_sh.at[pl.ds(row0, ROWS_PER_TILE)],
                        out_hbm.at[pl.ds(cid * N_NODES + row0, ROWS_PER_TILE)])

        @pl.when(sid == NS - 1)
        def _():
            t0 = NS * ROWS_PER_TILE
            pltpu.sync_copy(agg_sh.at[pl.ds(t0, TAIL_ROWS)],
                            out_hbm.at[pl.ds(cid * N_NODES + t0, TAIL_ROWS)])

    return k(node_msgs, src_idx, dst_idx, edge_part)


E_SPLITS = (128000, 192000)


def _final_tc(node_feats, aggs_list, W_out1T, W_out2T, b_out):
    BLK = 2000
    NB = N_NODES // BLK
    n_parts = 2 * len(aggs_list)

    def body(x_ref, *refs):
        a_refs = refs[:n_parts]
        w1_ref, w2_ref, b_ref, o_ref = refs[n_parts:]
        acc = jnp.dot(x_ref[...], w1_ref[...],
                      preferred_element_type=jnp.float32)
        agg = a_refs[0][...]
        for a in a_refs[1:]:
            agg = agg + a[...]
        acc = acc + jnp.dot(agg, w2_ref[...],
                            preferred_element_type=jnp.float32)
        o_ref[...] = jnp.maximum(acc + b_ref[...], 0.0)

    agg_specs = []
    agg_args = []
    for aggs in aggs_list:
        agg_specs.append(pl.BlockSpec((BLK, D), lambda i: (i, 0)))
        agg_specs.append(pl.BlockSpec((BLK, D), lambda i: (i + NB, 0)))
        agg_args.extend([aggs, aggs])

    return pl.pallas_call(
        body,
        grid=(NB,),
        in_specs=[pl.BlockSpec((BLK, D), lambda i: (i, 0))] + agg_specs +
                 [pl.BlockSpec((D, D), lambda i: (0, 0)),
                  pl.BlockSpec((D, D), lambda i: (0, 0)),
                  pl.BlockSpec((1, D), lambda i: (0, 0))],
        out_specs=pl.BlockSpec((BLK, D), lambda i: (i, 0)),
        out_shape=jax.ShapeDtypeStruct((N_NODES, D), jnp.float32),
    )(node_feats, *agg_args, W_out1T, W_out2T, b_out.reshape(1, D))


def kernel(node_feats, edge_index, edge_feats, W_msg, b_msg, W_out, b_out):
    edge_index = edge_index.astype(jnp.int32)
    src = edge_index[0]
    dst = edge_index[1]
    W_nodeT = W_msg[:, :D].T
    W_edgeT = W_msg[:, D:].T
    W_out1T = W_out[:, :D].T
    W_out2T = W_out[:, D:].T

    node_msgs = _node_msgs_tc(node_feats, W_nodeT)
    ef_t = edge_feats.T
    aggs_list = []
    off = 0
    for n_e in E_SPLITS:
        ep = _edge_part_tc(ef_t, W_edgeT, b_msg, off, n_e)
        aggs_list.append(
            _sc_aggregate(node_msgs, src, dst, ep, off, n_e // (NC * NS)))
        off += n_e
    return _final_tc(node_feats, aggs_list, W_out1T, W_out2T, b_out)
